# baseline (speedup 1.0000x reference)
import functools

import jax
import jax.numpy as jnp
from jax import lax
from jax.experimental import pallas as pl
from jax.experimental.pallas import tpu as pltpu
from jax.experimental.pallas import tpu_sc as plsc

N_PAD = 102400
LANES = 16
NCORES = 2
NSUB = 16
NW = NCORES * NSUB
CHUNK = 1024
KROWS = CHUNK // 128
C3 = 512
KR3 = C3 // 128
SLICE = N_PAD // NSUB
BN = 2048


def _leaky(q):
    return jnp.where(q >= 0.0, q, 0.2 * q)


def _p1_body(iters_pw, as_hbm, ad_hbm, src_hbm, dst_hbm, w_hbm, dpart_hbm,
             sh_as, sh_ad, sh_den, idx_s, idx_d, vals_s, vals_d, wbuf, zbuf,
             sem_ld, sem_g, sem_sc, sem_w):
    cid = lax.axis_index("c")
    sid = lax.axis_index("s")
    wid = cid * NSUB + sid
    n = iters_pw

    def _z(i, _):
        zbuf[pl.ds(i * LANES, LANES)] = jnp.zeros((LANES,), jnp.float32)
        return 0
    lax.fori_loop(0, SLICE // LANES, _z, 0)
    sl = pl.ds(sid * SLICE, SLICE)
    pltpu.sync_copy(as_hbm.at[sl], sh_as.at[sl])
    pltpu.sync_copy(ad_hbm.at[sl], sh_ad.at[sl])
    pltpu.sync_copy(zbuf, sh_den.at[sl])
    plsc.subcore_barrier()

    def _loads(c, b):
        r0 = (wid * n + c) * KROWS
        pltpu.async_copy(src_hbm.at[pl.ds(r0, KROWS)], idx_s.at[b], sem_ld)
        pltpu.async_copy(dst_hbm.at[pl.ds(r0, KROWS)], idx_d.at[b], sem_ld)

    def _drain_loads(b):
        pltpu.make_async_copy(src_hbm.at[pl.ds(0, KROWS)], idx_s.at[b],
                              sem_ld).wait()
        pltpu.make_async_copy(dst_hbm.at[pl.ds(0, KROWS)], idx_d.at[b],
                              sem_ld).wait()

    def _gathers(b, start):
        for j in range(KROWS):
            a = pltpu.async_copy if start else pltpu.make_async_copy
            d1 = a(sh_as.at[idx_s.at[b, j]],
                   vals_s.at[b, pl.ds(j * 128, 128)], sem_g)
            d2 = a(sh_ad.at[idx_d.at[b, j]],
                   vals_d.at[b, pl.ds(j * 128, 128)], sem_g)
            if not start:
                d1.wait()
                d2.wait()

    def _scatters(c, b, start):
        for j in range(KROWS):
            if start:
                pltpu.async_copy(wbuf.at[b, pl.ds(j * 128, 128)],
                                 sh_den.at[idx_d.at[b, j]], sem_sc, add=True)
            else:
                pltpu.make_async_copy(wbuf.at[b, pl.ds(j * 128, 128)],
                                      sh_den.at[idx_d.at[b, j]], sem_sc).wait()
        wslot = w_hbm.at[pl.ds((wid * n + c) * CHUNK, CHUNK)]
        if start:
            pltpu.async_copy(wbuf.at[b], wslot, sem_w)
        else:
            pltpu.make_async_copy(wbuf.at[b], wslot, sem_w).wait()

    _loads(0, 0)
    _drain_loads(0)
    _gathers(0, True)

    def _chunk(c, _):
        b = lax.rem(c, 2)
        nb = 1 - b

        @pl.when(c > 0)
        def _():
            _scatters(c - 1, nb, False)

        @pl.when(c < n - 1)
        def _():
            _loads(c + 1, nb)

        for j in range(KROWS):
            pltpu.make_async_copy(sh_as.at[idx_s.at[b, j]],
                                  vals_s.at[b, pl.ds(j * 128, 128)],
                                  sem_g).wait()
            pltpu.make_async_copy(sh_ad.at[idx_d.at[b, j]],
                                  vals_d.at[b, pl.ds(j * 128, 128)],
                                  sem_g).wait()

            def _cw(g, _, j=j):
                gsl = pl.ds(j * 128 + g * LANES, LANES)
                q = vals_s[b, gsl] + vals_d[b, gsl]
                wbuf[b, gsl] = jnp.exp(_leaky(q))
                return 0
            lax.fori_loop(0, 128 // LANES, _cw, 0)
            pltpu.async_copy(wbuf.at[b, pl.ds(j * 128, 128)],
                             sh_den.at[idx_d.at[b, j]], sem_sc, add=True)
        pltpu.async_copy(wbuf.at[b],
                         w_hbm.at[pl.ds((wid * n + c) * CHUNK, CHUNK)], sem_w)

        @pl.when(c < n - 1)
        def _():
            _drain_loads(nb)
            _gathers(nb, True)
        return 0
    lax.fori_loop(0, n, _chunk, 0)
    _scatters(n - 1, (n - 1) % 2, False)

    plsc.subcore_barrier()
    pltpu.sync_copy(sh_den.at[sl], dpart_hbm.at[cid, sl])


def _make_p1(e_pad):
    iters_pw = e_pad // (NW * CHUNK)
    rows = e_pad // 128
    mesh = plsc.VectorSubcoreMesh(core_axis_name="c", subcore_axis_name="s")
    return pl.kernel(
        functools.partial(_p1_body, iters_pw),
        compiler_params=pltpu.CompilerParams(use_tc_tiling_on_sc=False),
        out_type=(jax.ShapeDtypeStruct((e_pad,), jnp.float32),
                  jax.ShapeDtypeStruct((NCORES, N_PAD), jnp.float32)),
        mesh=mesh,
        scratch_types=[
            pltpu.VMEM_SHARED((N_PAD,), jnp.float32),
            pltpu.VMEM_SHARED((N_PAD,), jnp.float32),
            pltpu.VMEM_SHARED((N_PAD,), jnp.float32),
            pltpu.VMEM((2, KROWS, 128), jnp.int32),
            pltpu.VMEM((2, KROWS, 128), jnp.int32),
            pltpu.VMEM((2, CHUNK), jnp.float32),
            pltpu.VMEM((2, CHUNK), jnp.float32),
            pltpu.VMEM((2, CHUNK), jnp.float32),
            pltpu.VMEM((SLICE,), jnp.float32),
            pltpu.SemaphoreType.DMA,
            pltpu.SemaphoreType.DMA,
            pltpu.SemaphoreType.DMA,
            pltpu.SemaphoreType.DMA,
        ],
    )


def _pf_body(iters_pw, write_w, as_hbm, ad_hbm, hg_hbm, src_hbm, dst_hbm,
             *refs):
    if write_w:
        (w_hbm, dpart_hbm, part_hbm, sh_den, sh_acc, idx_s, idx_d, vals_s,
         vals_d, wbuf, rows, sem_ld, sem_ga, sem_gd, sem_gr, sem_sd,
         sem_sc, sem_w) = refs
    else:
        (dpart_hbm, part_hbm, sh_den, sh_acc, idx_s, idx_d, vals_s,
         vals_d, wbuf, rows, sem_ld, sem_ga, sem_gd, sem_gr, sem_sd,
         sem_sc) = refs
    cid = lax.axis_index("c")
    sid = lax.axis_index("s")
    wid = cid * NSUB + sid
    n = iters_pw

    def _z1(i, _):
        wbuf[0, pl.ds(i * LANES, LANES)] = jnp.zeros((LANES,), jnp.float32)
        return 0
    lax.fori_loop(0, C3 // LANES, _z1, 0)

    def _z2(i, _):
        rows[0, i, :] = jnp.zeros((LANES,), jnp.float32)
        return 0
    lax.fori_loop(0, C3, _z2, 0)

    def _zc1(t, _):
        pltpu.sync_copy(wbuf.at[0],
                        sh_den.at[pl.ds(sid * SLICE + t * C3, C3)])
        return 0
    lax.fori_loop(0, SLICE // C3, _zc1, 0)
    pltpu.sync_copy(wbuf.at[0, pl.ds(0, SLICE % C3)],
                    sh_den.at[pl.ds(sid * SLICE + (SLICE // C3) * C3,
                                    SLICE % C3)])

    def _zc2(t, _):
        pltpu.sync_copy(rows.at[0],
                        sh_acc.at[pl.ds(sid * SLICE + t * C3, C3)])
        return 0
    lax.fori_loop(0, SLICE // C3, _zc2, 0)
    pltpu.sync_copy(rows.at[0, pl.ds(0, SLICE % C3)],
                    sh_acc.at[pl.ds(sid * SLICE + (SLICE // C3) * C3,
                                    SLICE % C3)])
    plsc.subcore_barrier()

    def _loads(c, b):
        r0 = (wid * n + c) * KR3
        pltpu.async_copy(src_hbm.at[pl.ds(r0, KR3)], idx_s.at[b], sem_ld)
        pltpu.async_copy(dst_hbm.at[pl.ds(r0, KR3)], idx_d.at[b], sem_ld)

    def _drain_loads(b):
        pltpu.make_async_copy(src_hbm.at[pl.ds(0, KR3)], idx_s.at[b],
                              sem_ld).wait()
        pltpu.make_async_copy(dst_hbm.at[pl.ds(0, KR3)], idx_d.at[b],
                              sem_ld).wait()

    def _gathers(b, start):
        for j in range(KR3):
            jsl = pl.ds(j * 128, 128)
            for a_hbm, v, sem in ((as_hbm, vals_s, sem_ga),
                                  (ad_hbm, vals_d, sem_gd)):
                iref = idx_s if a_hbm is as_hbm else idx_d
                if start:
                    pltpu.async_copy(a_hbm.at[iref.at[b, j]], v.at[b, jsl],
                                     sem)
                else:
                    pltpu.make_async_copy(a_hbm.at[iref.at[b, j]],
                                          v.at[b, jsl], sem).wait()
            if start:
                pltpu.async_copy(hg_hbm.at[idx_s.at[b, j]], rows.at[b, jsl],
                                 sem_gr)

    def _scatters(c, b, start):
        for j in range(KR3):
            jsl = pl.ds(j * 128, 128)
            if start:
                pltpu.async_copy(wbuf.at[b, jsl], sh_den.at[idx_d.at[b, j]],
                                 sem_sd, add=True)
                pltpu.async_copy(rows.at[b, jsl], sh_acc.at[idx_d.at[b, j]],
                                 sem_sc, add=True)
            else:
                pltpu.make_async_copy(wbuf.at[b, jsl],
                                      sh_den.at[idx_d.at[b, j]], sem_sd).wait()
                pltpu.make_async_copy(rows.at[b, jsl],
                                      sh_acc.at[idx_d.at[b, j]], sem_sc).wait()
        if write_w:
            wslot = w_hbm.at[pl.ds((wid * n + c) * C3, C3)]
            if start:
                pltpu.async_copy(wbuf.at[b], wslot, sem_w)
            else:
                pltpu.make_async_copy(wbuf.at[b], wslot, sem_w).wait()

    _loads(0, 0)
    _drain_loads(0)
    _gathers(0, True)

    def _chunk(c, _):
        b = lax.rem(c, 2)
        nb = 1 - b

        @pl.when(c > 0)
        def _():
            _scatters(c - 1, nb, False)

        @pl.when(c < n - 1)
        def _():
            _loads(c + 1, nb)

        for j in range(KR3):
            jsl = pl.ds(j * 128, 128)
            pltpu.make_async_copy(as_hbm.at[idx_s.at[b, j]],
                                  vals_s.at[b, jsl], sem_ga).wait()
            pltpu.make_async_copy(ad_hbm.at[idx_d.at[b, j]],
                                  vals_d.at[b, jsl], sem_gd).wait()

            def _cw(g, _, j=j):
                gsl = pl.ds(j * 128 + g * LANES, LANES)
                q = vals_s[b, gsl] + vals_d[b, gsl]
                wbuf[b, gsl] = jnp.exp(_leaky(q))
                return 0
            lax.fori_loop(0, 128 // LANES, _cw, 0)
            pltpu.async_copy(wbuf.at[b, jsl], sh_den.at[idx_d.at[b, j]],
                             sem_sd, add=True)

            pltpu.make_async_copy(hg_hbm.at[idx_s.at[b, j]], rows.at[b, jsl],
                                  sem_gr).wait()

            def _scale(g, _, j=j):
                coef = wbuf[b, pl.ds(j * 128 + g * LANES, LANES)]
                for r in range(LANES):
                    row = j * 128 + g * LANES + r
                    rows[b, row, :] = rows[b, row, :] * coef[r]
                return 0
            lax.fori_loop(0, 128 // LANES, _scale, 0)
            pltpu.async_copy(rows.at[b, jsl], sh_acc.at[idx_d.at[b, j]],
                             sem_sc, add=True)
        if write_w:
            pltpu.async_copy(wbuf.at[b],
                             w_hbm.at[pl.ds((wid * n + c) * C3, C3)], sem_w)

        @pl.when(c < n - 1)
        def _():
            _drain_loads(nb)
            _gathers(nb, True)
        return 0
    lax.fori_loop(0, n, _chunk, 0)
    _scatters(n - 1, (n - 1) % 2, False)

    plsc.subcore_barrier()
    sl = pl.ds(sid * SLICE, SLICE)
    pltpu.sync_copy(sh_den.at[sl], dpart_hbm.at[cid, sl])
    pltpu.sync_copy(sh_acc.at[sl], part_hbm.at[cid, sl])


def _make_pf(e_pad, write_w):
    iters_pw = e_pad // (NW * C3)
    mesh = plsc.VectorSubcoreMesh(core_axis_name="c", subcore_axis_name="s")
    outs = [jax.ShapeDtypeStruct((NCORES, N_PAD), jnp.float32),
            jax.ShapeDtypeStruct((NCORES, N_PAD, LANES), jnp.float32)]
    if write_w:
        outs = [jax.ShapeDtypeStruct((e_pad,), jnp.float32)] + outs
    sems = [pltpu.SemaphoreType.DMA] * (7 if write_w else 6)
    return pl.kernel(
        functools.partial(_pf_body, iters_pw, write_w),
        compiler_params=pltpu.CompilerParams(use_tc_tiling_on_sc=False),
        out_type=tuple(outs),
        mesh=mesh,
        scratch_types=[
            pltpu.VMEM_SHARED((N_PAD,), jnp.float32),
            pltpu.VMEM_SHARED((N_PAD, LANES), jnp.float32),
            pltpu.VMEM((2, KR3, 128), jnp.int32),
            pltpu.VMEM((2, KR3, 128), jnp.int32),
            pltpu.VMEM((2, C3), jnp.float32),
            pltpu.VMEM((2, C3), jnp.float32),
            pltpu.VMEM((2, C3), jnp.float32),
            pltpu.VMEM((2, C3, LANES), jnp.float32),
        ] + sems,
    )


def _p3_body(iters_pw, hg_hbm, src_hbm, dst_hbm, w_hbm, part_hbm,
             sh_acc, idx_s, idx_d, wbuf, rows, zbuf,
             sem_ld, sem_g, sem_sc):
    cid = lax.axis_index("c")
    sid = lax.axis_index("s")
    wid = cid * NSUB + sid
    n = iters_pw

    def _z(i, _):
        zbuf[i, :] = jnp.zeros((LANES,), jnp.float32)
        return 0
    lax.fori_loop(0, 128, _z, 0)
    for t in range(SLICE // 128):
        pltpu.sync_copy(zbuf, sh_acc.at[pl.ds(sid * SLICE + t * 128, 128)])
    plsc.subcore_barrier()

    def _loads(c, b):
        r0 = (wid * n + c) * KR3
        pltpu.async_copy(src_hbm.at[pl.ds(r0, KR3)], idx_s.at[b], sem_ld)
        pltpu.async_copy(dst_hbm.at[pl.ds(r0, KR3)], idx_d.at[b], sem_ld)
        pltpu.async_copy(w_hbm.at[pl.ds((wid * n + c) * C3, C3)],
                         wbuf.at[b], sem_ld)

    def _drain_loads(b):
        pltpu.make_async_copy(src_hbm.at[pl.ds(0, KR3)], idx_s.at[b],
                              sem_ld).wait()
        pltpu.make_async_copy(dst_hbm.at[pl.ds(0, KR3)], idx_d.at[b],
                              sem_ld).wait()
        pltpu.make_async_copy(w_hbm.at[pl.ds(0, C3)], wbuf.at[b],
                              sem_ld).wait()

    def _gathers(b, start):
        for j in range(KR3):
            a = pltpu.async_copy if start else pltpu.make_async_copy
            d = a(hg_hbm.at[idx_s.at[b, j]],
                  rows.at[b, pl.ds(j * 128, 128)], sem_g)
            if not start:
                d.wait()

    def _scatters(b, start):
        for j in range(KR3):
            if start:
                pltpu.async_copy(rows.at[b, pl.ds(j * 128, 128)],
                                 sh_acc.at[idx_d.at[b, j]], sem_sc, add=True)
            else:
                pltpu.make_async_copy(rows.at[b, pl.ds(j * 128, 128)],
                                      sh_acc.at[idx_d.at[b, j]], sem_sc).wait()

    _loads(0, 0)
    _drain_loads(0)
    _gathers(0, True)

    def _chunk(c, _):
        b = lax.rem(c, 2)
        nb = 1 - b

        @pl.when(c > 0)
        def _():
            _scatters(nb, False)

        @pl.when(c < n - 1)
        def _():
            _loads(c + 1, nb)

        for j in range(KR3):
            pltpu.make_async_copy(hg_hbm.at[idx_s.at[b, j]],
                                  rows.at[b, pl.ds(j * 128, 128)],
                                  sem_g).wait()

            def _scale(g, _, j=j):
                coef = wbuf[b, pl.ds(j * 128 + g * LANES, LANES)]
                for r in range(LANES):
                    row = j * 128 + g * LANES + r
                    rows[b, row, :] = rows[b, row, :] * coef[r]
                return 0
            lax.fori_loop(0, 128 // LANES, _scale, 0)
            pltpu.async_copy(rows.at[b, pl.ds(j * 128, 128)],
                             sh_acc.at[idx_d.at[b, j]], sem_sc, add=True)

        @pl.when(c < n - 1)
        def _():
            _drain_loads(nb)
            _gathers(nb, True)
        return 0
    lax.fori_loop(0, n, _chunk, 0)
    _scatters((n - 1) % 2, False)

    plsc.subcore_barrier()
    pltpu.sync_copy(sh_acc.at[pl.ds(sid * SLICE, SLICE)],
                    part_hbm.at[cid, pl.ds(sid * SLICE, SLICE)])


def _make_p3(e_pad):
    iters_pw = e_pad // (NW * C3)
    mesh = plsc.VectorSubcoreMesh(core_axis_name="c", subcore_axis_name="s")
    return pl.kernel(
        functools.partial(_p3_body, iters_pw),
        compiler_params=pltpu.CompilerParams(use_tc_tiling_on_sc=False),
        out_type=jax.ShapeDtypeStruct((NCORES, N_PAD, LANES), jnp.float32),
        mesh=mesh,
        scratch_types=[
            pltpu.VMEM_SHARED((N_PAD, LANES), jnp.float32),
            pltpu.VMEM((2, KR3, 128), jnp.int32),
            pltpu.VMEM((2, KR3, 128), jnp.int32),
            pltpu.VMEM((2, C3), jnp.float32),
            pltpu.VMEM((2, C3, LANES), jnp.float32),
            pltpu.VMEM((128, LANES), jnp.float32),
            pltpu.SemaphoreType.DMA,
            pltpu.SemaphoreType.DMA,
            pltpu.SemaphoreType.DMA,
        ],
    )


def _t1_body(x_ref, w1t_ref, a1s_ref, a1d_ref, h_ref, as_ref, ad_ref, ws_ref):
    h = jnp.dot(x_ref[...], w1t_ref[...], preferred_element_type=jnp.float32)
    h_ref[...] = h
    asv = jnp.dot(h, a1s_ref[...], preferred_element_type=jnp.float32)
    adv = jnp.dot(h, a1d_ref[...], preferred_element_type=jnp.float32)
    as_ref[...] = asv
    ad_ref[...] = adv
    ws_ref[...] = jnp.exp(_leaky(asv + adv))


def _t2_body(p_ref, d_ref, ws_ref, h1_ref, w2t_ref, a2s_ref, a2d_ref, b1_ref,
             hg0_ref, hg1_ref, hg2_ref, hg3_ref, as_ref, ad_ref, ws2_ref):
    ws = ws_ref[...]
    denom = d_ref[0] + d_ref[1] + ws
    inv = 1.0 / (denom + 1e-16)
    g = (p_ref[0] + p_ref[1] + ws * h1_ref[...]) * inv + b1_ref[...]
    h2in = jnp.maximum(g, 0.0)
    h2 = jnp.dot(h2in, w2t_ref[...], preferred_element_type=jnp.float32)
    hg0_ref[...] = h2[:, 0:16]
    hg1_ref[...] = h2[:, 16:32]
    hg2_ref[...] = h2[:, 32:48]
    hg3_ref[...] = h2[:, 48:64]
    asv = jnp.dot(h2, a2s_ref[...], preferred_element_type=jnp.float32)
    adv = jnp.dot(h2, a2d_ref[...], preferred_element_type=jnp.float32)
    as_ref[...] = asv
    ad_ref[...] = adv
    ws2_ref[...] = jnp.exp(_leaky(asv + adv))


def _t3_body(p0_ref, p1_ref, p2_ref, p3_ref, d_ref, ws_ref,
             hg0_ref, hg1_ref, hg2_ref, hg3_ref, wlt_ref, b2_ref, bl_ref,
             out_ref):
    ws = ws_ref[...]
    inv = 1.0 / (d_ref[0] + d_ref[1] + ws + 1e-16)
    rowsum = jnp.concatenate(
        [p[0] + p[1] for p in (p0_ref, p1_ref, p2_ref, p3_ref)], axis=1)
    h2 = jnp.concatenate(
        [hg0_ref[...], hg1_ref[...], hg2_ref[...], hg3_ref[...]], axis=1)
    gat = (rowsum + ws * h2) * inv + b2_ref[...]
    out_ref[...] = jnp.dot(gat, wlt_ref[...],
                           preferred_element_type=jnp.float32) + bl_ref[...]


def _row_spec(feat):
    return pl.BlockSpec((BN, feat), lambda i: (i, 0))


def _full_spec(shape):
    return pl.BlockSpec(shape, lambda i: tuple(0 for _ in shape))


_GRID = N_PAD // BN

_t1 = pl.pallas_call(
    _t1_body,
    grid=(_GRID,),
    in_specs=[_row_spec(11), _full_spec((11, 16)), _full_spec((16, 1)),
              _full_spec((16, 1))],
    out_specs=[_row_spec(16), _row_spec(1), _row_spec(1), _row_spec(1)],
    out_shape=[jax.ShapeDtypeStruct((N_PAD, 16), jnp.float32),
               jax.ShapeDtypeStruct((N_PAD, 1), jnp.float32),
               jax.ShapeDtypeStruct((N_PAD, 1), jnp.float32),
               jax.ShapeDtypeStruct((N_PAD, 1), jnp.float32)],
)

_t2 = pl.pallas_call(
    _t2_body,
    grid=(_GRID,),
    in_specs=[pl.BlockSpec((2, BN, 16), lambda i: (0, i, 0)),
              pl.BlockSpec((2, BN, 1), lambda i: (0, i, 0)),
              _row_spec(1), _row_spec(16), _full_spec((16, 64)),
              _full_spec((64, 1)), _full_spec((64, 1)), _full_spec((1, 16))],
    out_specs=[_row_spec(16), _row_spec(16), _row_spec(16), _row_spec(16),
               _row_spec(1), _row_spec(1), _row_spec(1)],
    out_shape=[jax.ShapeDtypeStruct((N_PAD, 16), jnp.float32),
               jax.ShapeDtypeStruct((N_PAD, 16), jnp.float32),
               jax.ShapeDtypeStruct((N_PAD, 16), jnp.float32),
               jax.ShapeDtypeStruct((N_PAD, 16), jnp.float32),
               jax.ShapeDtypeStruct((N_PAD, 1), jnp.float32),
               jax.ShapeDtypeStruct((N_PAD, 1), jnp.float32),
               jax.ShapeDtypeStruct((N_PAD, 1), jnp.float32)],
)

_part_spec = pl.BlockSpec((2, BN, 16), lambda i: (0, i, 0))

_t3 = pl.pallas_call(
    _t3_body,
    grid=(_GRID,),
    in_specs=[_part_spec, _part_spec, _part_spec, _part_spec,
              pl.BlockSpec((2, BN, 1), lambda i: (0, i, 0)),
              _row_spec(1),
              _row_spec(16), _row_spec(16), _row_spec(16), _row_spec(16),
              _full_spec((64, 40)),
              _full_spec((1, 64)), _full_spec((1, 40))],
    out_specs=_row_spec(40),
    out_shape=jax.ShapeDtypeStruct((N_PAD, 40), jnp.float32),
)


def kernel(x, edge_index, batch, W1, a1_src, a1_dst, b1,
           W2, a2_src, a2_dst, b2, Wl, bl):
    del batch
    n = x.shape[0]
    e = edge_index.shape[1]
    e_pad = ((e + NW * CHUNK - 1) // (NW * CHUNK)) * (NW * CHUNK)
    rows = e_pad // 128

    xp = jnp.pad(x, ((0, N_PAD - n), (0, 0)))
    pad_idx = jnp.full((e_pad - e,), n, jnp.int32)
    src = jnp.concatenate([edge_index[0], pad_idx]).reshape(rows, 128)
    dst = jnp.concatenate([edge_index[1], pad_idx]).reshape(rows, 128)

    pf1 = _make_pf(e_pad, write_w=False)
    pf2 = _make_pf(e_pad, write_w=True)
    p3 = _make_p3(e_pad)

    h1, as1, ad1, ws1 = _t1(xp, W1.T, a1_src.reshape(16, 1),
                            a1_dst.reshape(16, 1))

    dpart1, part1 = pf1(as1.reshape(N_PAD), ad1.reshape(N_PAD), h1, src, dst)

    hg0, hg1, hg2, hg3, as2, ad2, ws2 = _t2(
        part1, dpart1.reshape(NCORES, N_PAD, 1), ws1, h1,
        W2.T, a2_src.reshape(64, 1), a2_dst.reshape(64, 1), b1.reshape(1, 16))

    w_e2, dpart2, part_g0 = pf2(as2.reshape(N_PAD), ad2.reshape(N_PAD), hg0,
                                src, dst)
    parts = [part_g0] + [p3(hg, src, dst, w_e2) for hg in (hg1, hg2, hg3)]

    out = _t3(parts[0], parts[1], parts[2], parts[3],
              dpart2.reshape(NCORES, N_PAD, 1), ws2,
              hg0, hg1, hg2, hg3, Wl.T, b2.reshape(1, 64), bl.reshape(1, 40))
    return out[:n]


# speedup vs baseline: 1.0782x; 1.0782x over previous
import functools

import jax
import jax.numpy as jnp
from jax import lax
from jax.experimental import pallas as pl
from jax.experimental.pallas import tpu as pltpu
from jax.experimental.pallas import tpu_sc as plsc

N_PAD = 102400
LANES = 16
NCORES = 2
NSUB = 16
NW = NCORES * NSUB
CHUNK = 1024
KROWS = CHUNK // 128
C3 = 512
KR3 = C3 // 128
SLICE = N_PAD // NSUB
BN = 2048


def _leaky(q):
    return jnp.where(q >= 0.0, q, 0.2 * q)


def _p1_body(iters_pw, as_hbm, ad_hbm, src_hbm, dst_hbm, w_hbm, dpart_hbm,
             sh_as, sh_ad, sh_den, idx_s, idx_d, vals_s, vals_d, wbuf, zbuf,
             sem_ld, sem_g, sem_sc, sem_w):
    cid = lax.axis_index("c")
    sid = lax.axis_index("s")
    wid = cid * NSUB + sid
    n = iters_pw

    def _z(i, _):
        zbuf[pl.ds(i * LANES, LANES)] = jnp.zeros((LANES,), jnp.float32)
        return 0
    lax.fori_loop(0, SLICE // LANES, _z, 0)
    sl = pl.ds(sid * SLICE, SLICE)
    pltpu.sync_copy(as_hbm.at[sl], sh_as.at[sl])
    pltpu.sync_copy(ad_hbm.at[sl], sh_ad.at[sl])
    pltpu.sync_copy(zbuf, sh_den.at[sl])
    plsc.subcore_barrier()

    def _loads(c, b):
        r0 = (wid * n + c) * KROWS
        pltpu.async_copy(src_hbm.at[pl.ds(r0, KROWS)], idx_s.at[b], sem_ld)
        pltpu.async_copy(dst_hbm.at[pl.ds(r0, KROWS)], idx_d.at[b], sem_ld)

    def _drain_loads(b):
        pltpu.make_async_copy(src_hbm.at[pl.ds(0, KROWS)], idx_s.at[b],
                              sem_ld).wait()
        pltpu.make_async_copy(dst_hbm.at[pl.ds(0, KROWS)], idx_d.at[b],
                              sem_ld).wait()

    def _gathers(b, start):
        for j in range(KROWS):
            a = pltpu.async_copy if start else pltpu.make_async_copy
            d1 = a(sh_as.at[idx_s.at[b, j]],
                   vals_s.at[b, pl.ds(j * 128, 128)], sem_g)
            d2 = a(sh_ad.at[idx_d.at[b, j]],
                   vals_d.at[b, pl.ds(j * 128, 128)], sem_g)
            if not start:
                d1.wait()
                d2.wait()

    def _scatters(c, b, start):
        for j in range(KROWS):
            if start:
                pltpu.async_copy(wbuf.at[b, pl.ds(j * 128, 128)],
                                 sh_den.at[idx_d.at[b, j]], sem_sc, add=True)
            else:
                pltpu.make_async_copy(wbuf.at[b, pl.ds(j * 128, 128)],
                                      sh_den.at[idx_d.at[b, j]], sem_sc).wait()
        wslot = w_hbm.at[pl.ds((wid * n + c) * CHUNK, CHUNK)]
        if start:
            pltpu.async_copy(wbuf.at[b], wslot, sem_w)
        else:
            pltpu.make_async_copy(wbuf.at[b], wslot, sem_w).wait()

    _loads(0, 0)
    _drain_loads(0)
    _gathers(0, True)

    def _chunk(c, _):
        b = lax.rem(c, 2)
        nb = 1 - b

        @pl.when(c > 0)
        def _():
            _scatters(c - 1, nb, False)

        @pl.when(c < n - 1)
        def _():
            _loads(c + 1, nb)

        for j in range(KROWS):
            pltpu.make_async_copy(sh_as.at[idx_s.at[b, j]],
                                  vals_s.at[b, pl.ds(j * 128, 128)],
                                  sem_g).wait()
            pltpu.make_async_copy(sh_ad.at[idx_d.at[b, j]],
                                  vals_d.at[b, pl.ds(j * 128, 128)],
                                  sem_g).wait()

            def _cw(g, _, j=j):
                gsl = pl.ds(j * 128 + g * LANES, LANES)
                q = vals_s[b, gsl] + vals_d[b, gsl]
                wbuf[b, gsl] = jnp.exp(_leaky(q))
                return 0
            lax.fori_loop(0, 128 // LANES, _cw, 0)
            pltpu.async_copy(wbuf.at[b, pl.ds(j * 128, 128)],
                             sh_den.at[idx_d.at[b, j]], sem_sc, add=True)
        pltpu.async_copy(wbuf.at[b],
                         w_hbm.at[pl.ds((wid * n + c) * CHUNK, CHUNK)], sem_w)

        @pl.when(c < n - 1)
        def _():
            _drain_loads(nb)
            _gathers(nb, True)
        return 0
    lax.fori_loop(0, n, _chunk, 0)
    _scatters(n - 1, (n - 1) % 2, False)

    plsc.subcore_barrier()
    pltpu.sync_copy(sh_den.at[sl], dpart_hbm.at[cid, sl])


def _make_p1(e_pad):
    iters_pw = e_pad // (NW * CHUNK)
    rows = e_pad // 128
    mesh = plsc.VectorSubcoreMesh(core_axis_name="c", subcore_axis_name="s")
    return pl.kernel(
        functools.partial(_p1_body, iters_pw),
        compiler_params=pltpu.CompilerParams(use_tc_tiling_on_sc=False),
        out_type=(jax.ShapeDtypeStruct((e_pad,), jnp.float32),
                  jax.ShapeDtypeStruct((NCORES, N_PAD), jnp.float32)),
        mesh=mesh,
        scratch_types=[
            pltpu.VMEM_SHARED((N_PAD,), jnp.float32),
            pltpu.VMEM_SHARED((N_PAD,), jnp.float32),
            pltpu.VMEM_SHARED((N_PAD,), jnp.float32),
            pltpu.VMEM((2, KROWS, 128), jnp.int32),
            pltpu.VMEM((2, KROWS, 128), jnp.int32),
            pltpu.VMEM((2, CHUNK), jnp.float32),
            pltpu.VMEM((2, CHUNK), jnp.float32),
            pltpu.VMEM((2, CHUNK), jnp.float32),
            pltpu.VMEM((SLICE,), jnp.float32),
            pltpu.SemaphoreType.DMA,
            pltpu.SemaphoreType.DMA,
            pltpu.SemaphoreType.DMA,
            pltpu.SemaphoreType.DMA,
        ],
    )


def _p3_body(iters_pw, hg_hbm, src_hbm, dst_hbm, w_hbm, part_hbm,
             sh_acc, idx_s, idx_d, wbuf, rows, zbuf,
             sem_ld, sem_g, sem_sc):
    cid = lax.axis_index("c")
    sid = lax.axis_index("s")
    wid = cid * NSUB + sid
    n = iters_pw

    def _z(i, _):
        zbuf[i, :] = jnp.zeros((LANES,), jnp.float32)
        return 0
    lax.fori_loop(0, 128, _z, 0)
    for t in range(SLICE // 128):
        pltpu.sync_copy(zbuf, sh_acc.at[pl.ds(sid * SLICE + t * 128, 128)])
    plsc.subcore_barrier()

    def _loads(c, b):
        r0 = (wid * n + c) * KR3
        pltpu.async_copy(src_hbm.at[pl.ds(r0, KR3)], idx_s.at[b], sem_ld)
        pltpu.async_copy(dst_hbm.at[pl.ds(r0, KR3)], idx_d.at[b], sem_ld)
        pltpu.async_copy(w_hbm.at[pl.ds((wid * n + c) * C3, C3)],
                         wbuf.at[b], sem_ld)

    def _drain_loads(b):
        pltpu.make_async_copy(src_hbm.at[pl.ds(0, KR3)], idx_s.at[b],
                              sem_ld).wait()
        pltpu.make_async_copy(dst_hbm.at[pl.ds(0, KR3)], idx_d.at[b],
                              sem_ld).wait()
        pltpu.make_async_copy(w_hbm.at[pl.ds(0, C3)], wbuf.at[b],
                              sem_ld).wait()

    def _gathers(b, start):
        for j in range(KR3):
            a = pltpu.async_copy if start else pltpu.make_async_copy
            d = a(hg_hbm.at[idx_s.at[b, j]],
                  rows.at[b, pl.ds(j * 128, 128)], sem_g)
            if not start:
                d.wait()

    def _scatters(b, start):
        for j in range(KR3):
            if start:
                pltpu.async_copy(rows.at[b, pl.ds(j * 128, 128)],
                                 sh_acc.at[idx_d.at[b, j]], sem_sc, add=True)
            else:
                pltpu.make_async_copy(rows.at[b, pl.ds(j * 128, 128)],
                                      sh_acc.at[idx_d.at[b, j]], sem_sc).wait()

    _loads(0, 0)
    _drain_loads(0)
    _gathers(0, True)

    def _chunk(c, _):
        b = lax.rem(c, 2)
        nb = 1 - b

        @pl.when(c > 0)
        def _():
            _scatters(nb, False)

        @pl.when(c < n - 1)
        def _():
            _loads(c + 1, nb)

        for j in range(KR3):
            pltpu.make_async_copy(hg_hbm.at[idx_s.at[b, j]],
                                  rows.at[b, pl.ds(j * 128, 128)],
                                  sem_g).wait()

            def _scale(g, _, j=j):
                coef = wbuf[b, pl.ds(j * 128 + g * LANES, LANES)]
                for r in range(LANES):
                    row = j * 128 + g * LANES + r
                    rows[b, row, :] = rows[b, row, :] * coef[r]
                return 0
            lax.fori_loop(0, 128 // LANES, _scale, 0)
            pltpu.async_copy(rows.at[b, pl.ds(j * 128, 128)],
                             sh_acc.at[idx_d.at[b, j]], sem_sc, add=True)

        @pl.when(c < n - 1)
        def _():
            _drain_loads(nb)
            _gathers(nb, True)
        return 0
    lax.fori_loop(0, n, _chunk, 0)
    _scatters((n - 1) % 2, False)

    plsc.subcore_barrier()
    pltpu.sync_copy(sh_acc.at[pl.ds(sid * SLICE, SLICE)],
                    part_hbm.at[cid, pl.ds(sid * SLICE, SLICE)])


def _make_p3(e_pad):
    iters_pw = e_pad // (NW * C3)
    mesh = plsc.VectorSubcoreMesh(core_axis_name="c", subcore_axis_name="s")
    return pl.kernel(
        functools.partial(_p3_body, iters_pw),
        compiler_params=pltpu.CompilerParams(use_tc_tiling_on_sc=False),
        out_type=jax.ShapeDtypeStruct((NCORES, N_PAD, LANES), jnp.float32),
        mesh=mesh,
        scratch_types=[
            pltpu.VMEM_SHARED((N_PAD, LANES), jnp.float32),
            pltpu.VMEM((2, KR3, 128), jnp.int32),
            pltpu.VMEM((2, KR3, 128), jnp.int32),
            pltpu.VMEM((2, C3), jnp.float32),
            pltpu.VMEM((2, C3, LANES), jnp.float32),
            pltpu.VMEM((128, LANES), jnp.float32),
            pltpu.SemaphoreType.DMA,
            pltpu.SemaphoreType.DMA,
            pltpu.SemaphoreType.DMA,
        ],
    )


def _t1_body(x_ref, w1t_ref, a1s_ref, a1d_ref, h_ref, as_ref, ad_ref, ws_ref):
    h = jnp.dot(x_ref[...], w1t_ref[...], preferred_element_type=jnp.float32)
    h_ref[...] = h
    asv = jnp.dot(h, a1s_ref[...], preferred_element_type=jnp.float32)
    adv = jnp.dot(h, a1d_ref[...], preferred_element_type=jnp.float32)
    as_ref[...] = asv
    ad_ref[...] = adv
    ws_ref[...] = jnp.exp(_leaky(asv + adv))


def _t2_body(p_ref, d_ref, ws_ref, h1_ref, w2t_ref, a2s_ref, a2d_ref, b1_ref,
             hg0_ref, hg1_ref, hg2_ref, hg3_ref, as_ref, ad_ref, ws2_ref):
    ws = ws_ref[...]
    denom = d_ref[0] + d_ref[1] + ws
    inv = 1.0 / (denom + 1e-16)
    g = (p_ref[0] + p_ref[1] + ws * h1_ref[...]) * inv + b1_ref[...]
    h2in = jnp.maximum(g, 0.0)
    h2 = jnp.dot(h2in, w2t_ref[...], preferred_element_type=jnp.float32)
    hg0_ref[...] = h2[:, 0:16]
    hg1_ref[...] = h2[:, 16:32]
    hg2_ref[...] = h2[:, 32:48]
    hg3_ref[...] = h2[:, 48:64]
    asv = jnp.dot(h2, a2s_ref[...], preferred_element_type=jnp.float32)
    adv = jnp.dot(h2, a2d_ref[...], preferred_element_type=jnp.float32)
    as_ref[...] = asv
    ad_ref[...] = adv
    ws2_ref[...] = jnp.exp(_leaky(asv + adv))


def _t3_body(p0_ref, p1_ref, p2_ref, p3_ref, d_ref, ws_ref,
             hg0_ref, hg1_ref, hg2_ref, hg3_ref, wlt_ref, b2_ref, bl_ref,
             out_ref):
    ws = ws_ref[...]
    inv = 1.0 / (d_ref[0] + d_ref[1] + ws + 1e-16)
    rowsum = jnp.concatenate(
        [p[0] + p[1] for p in (p0_ref, p1_ref, p2_ref, p3_ref)], axis=1)
    h2 = jnp.concatenate(
        [hg0_ref[...], hg1_ref[...], hg2_ref[...], hg3_ref[...]], axis=1)
    gat = (rowsum + ws * h2) * inv + b2_ref[...]
    out_ref[...] = jnp.dot(gat, wlt_ref[...],
                           preferred_element_type=jnp.float32) + bl_ref[...]


def _row_spec(feat):
    return pl.BlockSpec((BN, feat), lambda i: (i, 0))


def _full_spec(shape):
    return pl.BlockSpec(shape, lambda i: tuple(0 for _ in shape))


_GRID = N_PAD // BN

_t1 = pl.pallas_call(
    _t1_body,
    grid=(_GRID,),
    in_specs=[_row_spec(11), _full_spec((11, 16)), _full_spec((16, 1)),
              _full_spec((16, 1))],
    out_specs=[_row_spec(16), _row_spec(1), _row_spec(1), _row_spec(1)],
    out_shape=[jax.ShapeDtypeStruct((N_PAD, 16), jnp.float32),
               jax.ShapeDtypeStruct((N_PAD, 1), jnp.float32),
               jax.ShapeDtypeStruct((N_PAD, 1), jnp.float32),
               jax.ShapeDtypeStruct((N_PAD, 1), jnp.float32)],
)

_t2 = pl.pallas_call(
    _t2_body,
    grid=(_GRID,),
    in_specs=[pl.BlockSpec((2, BN, 16), lambda i: (0, i, 0)),
              pl.BlockSpec((2, BN, 1), lambda i: (0, i, 0)),
              _row_spec(1), _row_spec(16), _full_spec((16, 64)),
              _full_spec((64, 1)), _full_spec((64, 1)), _full_spec((1, 16))],
    out_specs=[_row_spec(16), _row_spec(16), _row_spec(16), _row_spec(16),
               _row_spec(1), _row_spec(1), _row_spec(1)],
    out_shape=[jax.ShapeDtypeStruct((N_PAD, 16), jnp.float32),
               jax.ShapeDtypeStruct((N_PAD, 16), jnp.float32),
               jax.ShapeDtypeStruct((N_PAD, 16), jnp.float32),
               jax.ShapeDtypeStruct((N_PAD, 16), jnp.float32),
               jax.ShapeDtypeStruct((N_PAD, 1), jnp.float32),
               jax.ShapeDtypeStruct((N_PAD, 1), jnp.float32),
               jax.ShapeDtypeStruct((N_PAD, 1), jnp.float32)],
)

_part_spec = pl.BlockSpec((2, BN, 16), lambda i: (0, i, 0))

_t3 = pl.pallas_call(
    _t3_body,
    grid=(_GRID,),
    in_specs=[_part_spec, _part_spec, _part_spec, _part_spec,
              pl.BlockSpec((2, BN, 1), lambda i: (0, i, 0)),
              _row_spec(1),
              _row_spec(16), _row_spec(16), _row_spec(16), _row_spec(16),
              _full_spec((64, 40)),
              _full_spec((1, 64)), _full_spec((1, 40))],
    out_specs=_row_spec(40),
    out_shape=jax.ShapeDtypeStruct((N_PAD, 40), jnp.float32),
)


def kernel(x, edge_index, batch, W1, a1_src, a1_dst, b1,
           W2, a2_src, a2_dst, b2, Wl, bl):
    del batch
    n = x.shape[0]
    e = edge_index.shape[1]
    e_pad = ((e + NW * CHUNK - 1) // (NW * CHUNK)) * (NW * CHUNK)
    rows = e_pad // 128

    xp = jnp.pad(x, ((0, N_PAD - n), (0, 0)))
    pad_idx = jnp.full((e_pad - e,), n, jnp.int32)
    src = jnp.concatenate([edge_index[0], pad_idx]).reshape(rows, 128)
    dst = jnp.concatenate([edge_index[1], pad_idx]).reshape(rows, 128)

    p1 = _make_p1(e_pad)
    p3 = _make_p3(e_pad)

    h1, as1, ad1, ws1 = _t1(xp, W1.T, a1_src.reshape(16, 1),
                            a1_dst.reshape(16, 1))

    w_e1, dpart1 = p1(as1.reshape(N_PAD), ad1.reshape(N_PAD), src, dst)
    part1 = p3(h1, src, dst, w_e1)

    hg0, hg1, hg2, hg3, as2, ad2, ws2 = _t2(
        part1, dpart1.reshape(NCORES, N_PAD, 1), ws1, h1,
        W2.T, a2_src.reshape(64, 1), a2_dst.reshape(64, 1), b1.reshape(1, 16))

    w_e2, dpart2 = p1(as2.reshape(N_PAD), ad2.reshape(N_PAD), src, dst)
    parts = [p3(hg, src, dst, w_e2) for hg in (hg0, hg1, hg2, hg3)]

    out = _t3(parts[0], parts[1], parts[2], parts[3],
              dpart2.reshape(NCORES, N_PAD, 1), ws2,
              hg0, hg1, hg2, hg3, Wl.T, b2.reshape(1, 64), bl.reshape(1, 40))
    return out[:n]
